# baseline (speedup 1.0000x reference)
import functools

import jax
import jax.numpy as jnp
from jax import lax
from jax.experimental import pallas as pl
from jax.experimental.pallas import tpu as pltpu
from jax.experimental.pallas import tpu_sc as plsc

_K = 8
_BPB = 4
_EPB = 8


def _router_kernel(x_ref, fc1W_ref, fc1b_ref, gateWT_ref, gatebT_ref,
                   h_ref, idx_ref, probs_ref):
    S = x_ref.shape[1]
    E = gateWT_ref.shape[0]
    g = pl.program_id(0)
    pieces = []
    for bi in range(_BPB):
        hb = jnp.dot(x_ref[bi], fc1W_ref[...],
                     preferred_element_type=jnp.float32) + fc1b_ref[...]
        h_ref[bi] = jnp.pad(hb, ((0, 0), (0, 96)))
        pieces.append(lax.dot_general(
            gateWT_ref[...], hb, (((1,), (1,)), ((), ())),
            preferred_element_type=jnp.float32))
    lT = jnp.concatenate(pieces, axis=0)
    lT = lT + jnp.concatenate([gatebT_ref[...]] * _BPB, axis=0)
    m = jnp.max(lT, axis=1, keepdims=True)
    ex = jnp.exp(lT - m)
    gpT = ex / jnp.sum(ex, axis=1, keepdims=True)
    lane = lax.broadcasted_iota(jnp.int32, gpT.shape, 1).astype(jnp.float32)
    work = gpT
    idx_cols = []
    probs_cols = []
    for _ in range(_K):
        mk = jnp.max(work, axis=1, keepdims=True)
        ik = jnp.min(jnp.where(work == mk, lane, float(S)),
                     axis=1, keepdims=True)
        sel = (lane == ik)
        idx_cols.append(ik)
        probs_cols.append(mk)
        work = jnp.where(sel, 0.0, work)
    probs_ref[...] = jnp.concatenate(probs_cols, axis=1)
    tok = jnp.concatenate(idx_cols, axis=1).astype(jnp.int32)
    bi_vec = lax.broadcasted_iota(jnp.int32, tok.shape, 0) // E
    idx_ref[...] = tok + S * (g * _BPB + bi_vec)


def _expert_kernel(ginp_ref, probs_ref, eW_ref, eb_ref, fc2W_ref, fc2b_ref,
                   fc3W_ref, fc3b_ref, W4r_ref, fc4b_ref,
                   fc5W_ref, fc5b_ref, fc6W_ref, fc6b_ref,
                   out_ref, acc_ref):
    g = pl.program_id(0)
    KW = 32 * _K
    rep = (lax.broadcasted_iota(jnp.int32, (_K, KW), 1) // 32
           == lax.broadcasted_iota(jnp.int32, (_K, KW), 0)).astype(jnp.float32)
    contrib = None
    for j in range(_EPB):
        probs = probs_ref[:, j, :]
        w = jnp.dot(probs, rep, preferred_element_type=jnp.float32)
        raw = ginp_ref[:, j, :]
        g256 = jnp.concatenate(
            [raw[:, k * 128:k * 128 + 32] for k in range(_K)], axis=1)
        inp = jnp.concatenate([g256 * w, probs], axis=1)
        p = jnp.sum(probs, axis=1, keepdims=True)
        out = jnp.dot(inp, eW_ref[j],
                      preferred_element_type=jnp.float32) + eb_ref[j]
        out = out * p
        moe = jnp.maximum(jnp.concatenate([out, probs], axis=1), 0.0)
        y = jnp.maximum(jnp.dot(moe, fc2W_ref[...],
                                preferred_element_type=jnp.float32)
                        + fc2b_ref[...], 0.0)
        y = jnp.maximum(jnp.dot(y, fc3W_ref[...],
                                preferred_element_type=jnp.float32)
                        + fc3b_ref[...], 0.0)
        c = jnp.dot(y, W4r_ref[j], preferred_element_type=jnp.float32)
        contrib = c if contrib is None else contrib + c

    @pl.when(g == 0)
    def _():
        acc_ref[...] = contrib

    @pl.when(g != 0)
    def _():
        acc_ref[...] = acc_ref[...] + contrib

    @pl.when(g == pl.num_programs(0) - 1)
    def _():
        z = jnp.maximum(acc_ref[...] + fc4b_ref[...], 0.0)
        z = jnp.maximum(jnp.dot(z, fc5W_ref[...],
                                preferred_element_type=jnp.float32)
                        + fc5b_ref[...], 0.0)
        out_ref[...] = jnp.dot(z, fc6W_ref[...],
                               preferred_element_type=jnp.float32) + fc6b_ref[...]


def _sc_gather(h_flat, idx_flat):
    N = idx_flat.shape[0]
    D = h_flat.shape[1]
    info = plsc.get_sparse_core_info()
    NW = info.num_cores * info.num_subcores
    CH = 128
    n_per_w = N // NW
    n_chunks = n_per_w // CH
    mesh = plsc.VectorSubcoreMesh(core_axis_name="c", subcore_axis_name="s")

    @functools.partial(
        pl.kernel, mesh=mesh,
        out_type=jax.ShapeDtypeStruct((N, D), jnp.float32),
        scratch_types=[
            pltpu.VMEM((CH,), jnp.int32),
            pltpu.VMEM((CH, D), jnp.float32),
            pltpu.SemaphoreType.DMA,
        ],
    )
    def k(h_hbm, idx_hbm, out_hbm, idx_v, rows_v, sem):
        wid = lax.axis_index("s") * info.num_cores + lax.axis_index("c")
        base = wid * n_per_w
        for c in range(n_chunks):
            off = base + c * CH
            pltpu.sync_copy(idx_hbm.at[pl.ds(off, CH)], idx_v)
            pltpu.async_copy(h_hbm.at[idx_v], rows_v, sem).wait()
            pltpu.sync_copy(rows_v, out_hbm.at[pl.ds(off, CH)])

    return k(h_flat, idx_flat)


@jax.jit
def kernel(x, fc1_W, fc1_b, gate_W, gate_b, expert_W, expert_b,
           fc2_W, fc2_b, fc3_W, fc3_b, fc4_W, fc4_b,
           fc5_W, fc5_b, fc6_W, fc6_b):
    B, S, TOK = x.shape
    E = gate_W.shape[1]
    IW = 32 * _K + _K
    O = expert_W.shape[-1]

    h, idx, probs = pl.pallas_call(
        _router_kernel,
        grid=(B // _BPB,),
        in_specs=[
            pl.BlockSpec((_BPB, S, TOK), lambda b: (b, 0, 0)),
            pl.BlockSpec((TOK, 32), lambda b: (0, 0)),
            pl.BlockSpec((1, 32), lambda b: (0, 0)),
            pl.BlockSpec((E, 32), lambda b: (0, 0)),
            pl.BlockSpec((E, 1), lambda b: (0, 0)),
        ],
        out_specs=[
            pl.BlockSpec((_BPB, S, 128), lambda b: (b, 0, 0)),
            pl.BlockSpec((_BPB * E, _K), lambda b: (b, 0)),
            pl.BlockSpec((_BPB * E, _K), lambda b: (b, 0)),
        ],
        out_shape=[
            jax.ShapeDtypeStruct((B, S, 128), jnp.float32),
            jax.ShapeDtypeStruct((B * E, _K), jnp.int32),
            jax.ShapeDtypeStruct((B * E, _K), jnp.float32),
        ],
    )(x, fc1_W, fc1_b.reshape(1, 32), gate_W.T, gate_b.reshape(E, 1))

    gathered = _sc_gather(h.reshape(B * S, 128), idx.reshape(B * E * _K))
    ginp = gathered.reshape(B, E, _K * 128)
    probs_bek = probs.reshape(B, E, _K)

    W4r = fc4_W.reshape(E, 128, 128)
    out = pl.pallas_call(
        _expert_kernel,
        grid=(E // _EPB,),
        in_specs=[
            pl.BlockSpec((B, _EPB, _K * 128), lambda g: (0, g, 0)),
            pl.BlockSpec((B, _EPB, _K), lambda g: (0, g, 0)),
            pl.BlockSpec((_EPB, IW, O), lambda g: (g, 0, 0)),
            pl.BlockSpec((_EPB, 1, O), lambda g: (g, 0, 0)),
            pl.BlockSpec((32, 128), lambda g: (0, 0)),
            pl.BlockSpec((1, 128), lambda g: (0, 0)),
            pl.BlockSpec((128, 128), lambda g: (0, 0)),
            pl.BlockSpec((1, 128), lambda g: (0, 0)),
            pl.BlockSpec((_EPB, 128, 128), lambda g: (g, 0, 0)),
            pl.BlockSpec((1, 128), lambda g: (0, 0)),
            pl.BlockSpec((128, 128), lambda g: (0, 0)),
            pl.BlockSpec((1, 128), lambda g: (0, 0)),
            pl.BlockSpec((128, 10), lambda g: (0, 0)),
            pl.BlockSpec((1, 10), lambda g: (0, 0)),
        ],
        out_specs=pl.BlockSpec((B, 10), lambda g: (0, 0)),
        out_shape=jax.ShapeDtypeStruct((B, 10), jnp.float32),
        scratch_shapes=[pltpu.VMEM((B, 128), jnp.float32)],
    )(ginp, probs_bek, expert_W, expert_b.reshape(E, 1, O), fc2_W,
      fc2_b.reshape(1, 128), fc3_W, fc3_b.reshape(1, 128), W4r,
      fc4_b.reshape(1, 128), fc5_W, fc5_b.reshape(1, 128),
      fc6_W, fc6_b.reshape(1, 10))
    return out


# speedup vs baseline: 1.0368x; 1.0368x over previous
import functools

import jax
import jax.numpy as jnp
from jax import lax
from jax.experimental import pallas as pl
from jax.experimental.pallas import tpu as pltpu
from jax.experimental.pallas import tpu_sc as plsc

_K = 8
_BPB = 8
_EPB = 8


def _router_kernel(x_ref, fc1W_ref, fc1b_ref, gateWT_ref, gatebT_ref,
                   h_ref, idx_ref, probs_ref):
    S = x_ref.shape[1]
    E = gateWT_ref.shape[0]
    g = pl.program_id(0)
    pieces = []
    for bi in range(_BPB):
        hb = jnp.dot(x_ref[bi], fc1W_ref[...],
                     preferred_element_type=jnp.float32) + fc1b_ref[...]
        h_ref[bi] = jnp.pad(hb, ((0, 0), (0, 96)))
        pieces.append(lax.dot_general(
            gateWT_ref[...], hb, (((1,), (1,)), ((), ())),
            preferred_element_type=jnp.float32))
    lT = jnp.concatenate(pieces, axis=0)
    lT = lT + jnp.concatenate([gatebT_ref[...]] * _BPB, axis=0)
    m = jnp.max(lT, axis=1, keepdims=True)
    ex = jnp.exp(lT - m)
    gpT = ex / jnp.sum(ex, axis=1, keepdims=True)
    lane = lax.broadcasted_iota(jnp.int32, gpT.shape, 1).astype(jnp.float32)
    work = gpT
    idx_cols = []
    probs_cols = []
    for _ in range(_K):
        mk = jnp.max(work, axis=1, keepdims=True)
        ik = jnp.min(jnp.where(work == mk, lane, float(S)),
                     axis=1, keepdims=True)
        sel = (lane == ik)
        idx_cols.append(ik)
        probs_cols.append(mk)
        work = jnp.where(sel, 0.0, work)
    probs_ref[...] = jnp.concatenate(probs_cols, axis=1)
    tok = jnp.concatenate(idx_cols, axis=1).astype(jnp.int32)
    bi_vec = lax.broadcasted_iota(jnp.int32, tok.shape, 0) // E
    idx_ref[...] = tok + S * (g * _BPB + bi_vec)


def _expert_kernel(ginp_ref, probs_ref, eW_ref, eb_ref, fc2W_ref, fc2b_ref,
                   fc3W_ref, fc3b_ref, W4r_ref, fc4b_ref,
                   fc5W_ref, fc5b_ref, fc6W_ref, fc6b_ref,
                   out_ref, acc_ref):
    g = pl.program_id(0)
    KW = 32 * _K
    rep = (lax.broadcasted_iota(jnp.int32, (_K, KW), 1) // 32
           == lax.broadcasted_iota(jnp.int32, (_K, KW), 0)).astype(jnp.float32)
    contrib = None
    for j in range(_EPB):
        probs = probs_ref[:, j, :]
        w = jnp.dot(probs, rep, preferred_element_type=jnp.float32)
        raw = ginp_ref[:, j, :]
        g256 = jnp.concatenate(
            [raw[:, k * 128:k * 128 + 32] for k in range(_K)], axis=1)
        inp = jnp.concatenate([g256 * w, probs], axis=1)
        p = jnp.sum(probs, axis=1, keepdims=True)
        out = jnp.dot(inp, eW_ref[j],
                      preferred_element_type=jnp.float32) + eb_ref[j]
        out = out * p
        moe = jnp.maximum(jnp.concatenate([out, probs], axis=1), 0.0)
        y = jnp.maximum(jnp.dot(moe, fc2W_ref[...],
                                preferred_element_type=jnp.float32)
                        + fc2b_ref[...], 0.0)
        y = jnp.maximum(jnp.dot(y, fc3W_ref[...],
                                preferred_element_type=jnp.float32)
                        + fc3b_ref[...], 0.0)
        c = jnp.dot(y, W4r_ref[j], preferred_element_type=jnp.float32)
        contrib = c if contrib is None else contrib + c

    @pl.when(g == 0)
    def _():
        acc_ref[...] = contrib

    @pl.when(g != 0)
    def _():
        acc_ref[...] = acc_ref[...] + contrib

    @pl.when(g == pl.num_programs(0) - 1)
    def _():
        z = jnp.maximum(acc_ref[...] + fc4b_ref[...], 0.0)
        z = jnp.maximum(jnp.dot(z, fc5W_ref[...],
                                preferred_element_type=jnp.float32)
                        + fc5b_ref[...], 0.0)
        out_ref[...] = jnp.dot(z, fc6W_ref[...],
                               preferred_element_type=jnp.float32) + fc6b_ref[...]


def _sc_gather(h_flat, idx_flat):
    N = idx_flat.shape[0]
    D = h_flat.shape[1]
    info = plsc.get_sparse_core_info()
    NW = info.num_cores * info.num_subcores
    CH = 128
    n_per_w = N // NW
    n_chunks = n_per_w // CH
    mesh = plsc.VectorSubcoreMesh(core_axis_name="c", subcore_axis_name="s")

    @functools.partial(
        pl.kernel, mesh=mesh,
        out_type=jax.ShapeDtypeStruct((N, D), jnp.float32),
        scratch_types=[
            pltpu.VMEM((CH,), jnp.int32),
            pltpu.VMEM((CH, D), jnp.float32),
            pltpu.SemaphoreType.DMA,
        ],
    )
    def k(h_hbm, idx_hbm, out_hbm, idx_v, rows_v, sem):
        wid = lax.axis_index("s") * info.num_cores + lax.axis_index("c")
        base = wid * n_per_w
        for c in range(n_chunks):
            off = base + c * CH
            pltpu.sync_copy(idx_hbm.at[pl.ds(off, CH)], idx_v)
            pltpu.async_copy(h_hbm.at[idx_v], rows_v, sem).wait()
            pltpu.sync_copy(rows_v, out_hbm.at[pl.ds(off, CH)])

    return k(h_flat, idx_flat)


@jax.jit
def kernel(x, fc1_W, fc1_b, gate_W, gate_b, expert_W, expert_b,
           fc2_W, fc2_b, fc3_W, fc3_b, fc4_W, fc4_b,
           fc5_W, fc5_b, fc6_W, fc6_b):
    B, S, TOK = x.shape
    E = gate_W.shape[1]
    IW = 32 * _K + _K
    O = expert_W.shape[-1]

    h, idx, probs = pl.pallas_call(
        _router_kernel,
        grid=(B // _BPB,),
        in_specs=[
            pl.BlockSpec((_BPB, S, TOK), lambda b: (b, 0, 0)),
            pl.BlockSpec((TOK, 32), lambda b: (0, 0)),
            pl.BlockSpec((1, 32), lambda b: (0, 0)),
            pl.BlockSpec((E, 32), lambda b: (0, 0)),
            pl.BlockSpec((E, 1), lambda b: (0, 0)),
        ],
        out_specs=[
            pl.BlockSpec((_BPB, S, 128), lambda b: (b, 0, 0)),
            pl.BlockSpec((_BPB * E, _K), lambda b: (b, 0)),
            pl.BlockSpec((_BPB * E, _K), lambda b: (b, 0)),
        ],
        out_shape=[
            jax.ShapeDtypeStruct((B, S, 128), jnp.float32),
            jax.ShapeDtypeStruct((B * E, _K), jnp.int32),
            jax.ShapeDtypeStruct((B * E, _K), jnp.float32),
        ],
    )(x, fc1_W, fc1_b.reshape(1, 32), gate_W.T, gate_b.reshape(E, 1))

    gathered = _sc_gather(h.reshape(B * S, 128), idx.reshape(B * E * _K))
    ginp = gathered.reshape(B, E, _K * 128)
    probs_bek = probs.reshape(B, E, _K)

    W4r = fc4_W.reshape(E, 128, 128)
    out = pl.pallas_call(
        _expert_kernel,
        grid=(E // _EPB,),
        in_specs=[
            pl.BlockSpec((B, _EPB, _K * 128), lambda g: (0, g, 0)),
            pl.BlockSpec((B, _EPB, _K), lambda g: (0, g, 0)),
            pl.BlockSpec((_EPB, IW, O), lambda g: (g, 0, 0)),
            pl.BlockSpec((_EPB, 1, O), lambda g: (g, 0, 0)),
            pl.BlockSpec((32, 128), lambda g: (0, 0)),
            pl.BlockSpec((1, 128), lambda g: (0, 0)),
            pl.BlockSpec((128, 128), lambda g: (0, 0)),
            pl.BlockSpec((1, 128), lambda g: (0, 0)),
            pl.BlockSpec((_EPB, 128, 128), lambda g: (g, 0, 0)),
            pl.BlockSpec((1, 128), lambda g: (0, 0)),
            pl.BlockSpec((128, 128), lambda g: (0, 0)),
            pl.BlockSpec((1, 128), lambda g: (0, 0)),
            pl.BlockSpec((128, 10), lambda g: (0, 0)),
            pl.BlockSpec((1, 10), lambda g: (0, 0)),
        ],
        out_specs=pl.BlockSpec((B, 10), lambda g: (0, 0)),
        out_shape=jax.ShapeDtypeStruct((B, 10), jnp.float32),
        scratch_shapes=[pltpu.VMEM((B, 128), jnp.float32)],
    )(ginp, probs_bek, expert_W, expert_b.reshape(E, 1, O), fc2_W,
      fc2_b.reshape(1, 128), fc3_W, fc3_b.reshape(1, 128), W4r,
      fc4_b.reshape(1, 128), fc5_W, fc5_b.reshape(1, 128),
      fc6_W, fc6_b.reshape(1, 10))
    return out
